# baseline (speedup 1.0000x reference)
import functools

import jax
import jax.numpy as jnp
import numpy as np
from jax import lax
from jax.experimental import pallas as pl
from jax.experimental.pallas import tpu as pltpu
from jax.experimental.pallas import tpu_sc as plsc

_B = 4096
_NT = 26
_V = 100000
_D = 32
_NI = _NT + 1

_S = 25088
_VC = 896
_NCQ = _S // _VC

_NW = 32
_PER_W = _NT * _B // _NW
_CHUNK = 128
_NCHUNK = _PER_W // _CHUNK


def _tr_body(i0, i1, i2, i3, out_ref):
    row = lax.broadcasted_iota(jnp.int32, (_D, 128), 0)
    col = lax.broadcasted_iota(jnp.int32, (_D, 128), 1)
    es = [(col == row + q * 32).astype(jnp.float32) for q in range(4)]
    c = pl.program_id(0)
    v3 = 3 * _S + c * _VC + lax.broadcasted_iota(jnp.int32, (_D, _VC), 1)
    ok3 = v3 < _V
    refs = (i0, i1, i2, i3)
    acc = None
    for q in range(4):
        x = refs[q][0]
        if q == 3:
            x = jnp.where(ok3, x, 0.0)
        y = lax.dot_general(x, es[q], (((0,), (0,)), ((), ())),
                            preferred_element_type=jnp.float32)
        acc = y if acc is None else acc + y
    out_ref[0] = acc


def _relayout_table(tt3):
    spec = lambda q: pl.BlockSpec(
        (1, _D, _VC), lambda c, t, q=q: (t, 0, q * _NCQ + c))
    out = pl.pallas_call(
        _tr_body,
        grid=(_NCQ, _NT),
        in_specs=[spec(0), spec(1), spec(2), spec(3)],
        out_specs=pl.BlockSpec((1, _VC, 128), lambda c, t: (t, c, 0)),
        out_shape=jax.ShapeDtypeStruct((_NT, _S, 128), jnp.float32),
        compiler_params=pltpu.CompilerParams(fuse_transposed_lhs_in_matmul=True),
    )(tt3, tt3, tt3, tt3)
    return out.reshape(_NT * _S, 128)


def _sc_body(table_hbm, idx_hbm, out_hbm, idx_v, buf0, buf1, sem, psem):
    wid = lax.axis_index("s") * 2 + lax.axis_index("c")
    base = wid * _PER_W
    pltpu.sync_copy(idx_hbm.at[wid], idx_v)
    bufs = (buf0, buf1)
    puts = [None, None]
    for j in range(_NCHUNK):
        s = j % 2
        if puts[s] is not None:
            puts[s].wait()
        pltpu.async_copy(table_hbm.at[idx_v.at[j]], bufs[s], sem).wait()
        puts[s] = pltpu.async_copy(
            bufs[s], out_hbm.at[pl.ds(base + j * _CHUNK, _CHUNK)], psem
        )
    for s in range(2):
        if puts[s] is not None:
            puts[s].wait()


def _sc_gather(table4, gidx_grouped):
    mesh = plsc.VectorSubcoreMesh(core_axis_name="c", subcore_axis_name="s")
    k = functools.partial(
        pl.kernel,
        mesh=mesh,
        out_type=jax.ShapeDtypeStruct((_NT * _B, 128), jnp.float32),
        scratch_types=[
            pltpu.VMEM((_NCHUNK, _CHUNK), jnp.int32),
            pltpu.VMEM((_CHUNK, 128), jnp.float32),
            pltpu.VMEM((_CHUNK, 128), jnp.float32),
            pltpu.SemaphoreType.DMA,
            pltpu.SemaphoreType.DMA,
        ],
        compiler_params=pltpu.CompilerParams(use_tc_tiling_on_sc=True),
    )(_sc_body)
    return k(table4, gidx_grouped)


_BLK = 512
_GRID = _B // _BLK

_DNUM_T = (((1,), (1,)), ((), ()))
_DNUM = (((1,), (0,)), ((), ()))


def _mm_t(w, x):
    return lax.dot_general(w, x, _DNUM_T, preferred_element_type=jnp.float32)


def _mm(w, x):
    return lax.dot_general(w, x, _DNUM, preferred_element_type=jnp.float32)


def _fw_body(dense_ref, grp_ref, qsel_ref,
             bw0, bb0, bw1, bb1, bw2, bb2,
             tw0, tb0, tw1, tb1, tw2, tb2,
             out_ref):
    x = dense_ref[...]
    h = jnp.maximum(_mm_t(bw0[...], x) + bb0[...], 0.0)
    h = jnp.maximum(_mm(bw1[...], h) + bb1[...], 0.0)
    xb = jnp.maximum(_mm(bw2[...], h) + bb2[...], 0.0)

    grpt = jnp.transpose(grp_ref[...], (0, 2, 1))
    qsel = qsel_ref[...]
    lyt = grpt[:, 0:32, :]
    for q in range(1, 4):
        m = (qsel == q)[:, None, :]
        lyt = jnp.where(m, grpt[:, q * 32:(q + 1) * 32, :], lyt)
    a = jnp.concatenate([xb[None], lyt], axis=0)

    zs = []
    for i in range(1, _NI):
        prod = a[:i] * a[i][None]
        zs.append(jnp.sum(prod, axis=1))
    r = jnp.concatenate([xb] + zs, axis=0)

    t = jnp.maximum(_mm(tw0[...], r) + tb0[...], 0.0)
    t = jnp.maximum(_mm(tw1[...], t) + tb1[...], 0.0)
    t = _mm(tw2[...], t) + tb2[...]
    out_ref[...] = jax.nn.sigmoid(t)


def _tc_forward(dense_x, grp, qsel, bots, tops):
    full = lambda shape: pl.BlockSpec(shape, lambda i: tuple(0 for _ in shape))
    in_specs = [
        pl.BlockSpec((_BLK, 13), lambda i: (i, 0)),
        pl.BlockSpec((_NT, _BLK, 128), lambda i: (0, i, 0)),
        pl.BlockSpec((_NT, _BLK), lambda i: (0, i)),
    ]
    args = [dense_x, grp, qsel]
    for w, b in bots + tops:
        in_specs += [full(w.shape), full((b.shape[0], 1))]
        args += [w, b.reshape(-1, 1)]
    out = pl.pallas_call(
        _fw_body,
        grid=(_GRID,),
        in_specs=in_specs,
        out_specs=pl.BlockSpec((1, _BLK), lambda i: (0, i)),
        out_shape=jax.ShapeDtypeStruct((1, _B), jnp.float32),
    )(*args)
    return out.reshape(_B, 1)


def kernel(dense_x, lS_o, lS_i, emb_tables,
           bot_w0, bot_b0, bot_w1, bot_b1, bot_w2, bot_b2,
           top_w0, top_b0, top_w1, top_b1, top_w2, top_b2):
    del lS_o
    tt3 = emb_tables.transpose(0, 2, 1)
    table4 = _relayout_table(tt3)
    qsel = lS_i // _S
    gidx = (jnp.arange(_NT, dtype=jnp.int32) * _S)[:, None] + lS_i % _S
    gidx_grouped = gidx.reshape(_NW, _NCHUNK, _CHUNK)
    grp = _sc_gather(table4, gidx_grouped)
    grp3 = grp.reshape(_NT, _B, 128)
    bots = [(bot_w0, bot_b0), (bot_w1, bot_b1), (bot_w2, bot_b2)]
    tops = [(top_w0, top_b0), (top_w1, top_b1), (top_w2, top_b2)]
    return _tc_forward(dense_x, grp3, qsel, bots, tops)


# speedup vs baseline: 1.7261x; 1.7261x over previous
import functools

import jax
import jax.numpy as jnp
import numpy as np
from jax import lax
from jax.experimental import pallas as pl
from jax.experimental.pallas import tpu as pltpu
from jax.experimental.pallas import tpu_sc as plsc

_B = 4096
_NT = 26
_V = 100000
_D = 32
_NI = _NT + 1

_S = 25088
_VC = 896
_NCQ = _S // _VC

_NW = 32
_PER_W = _NT * _B // _NW
_CHUNK = 128
_NCHUNK = _PER_W // _CHUNK


def _tr_body(i0, i1, i2, i3, out_ref):
    row = lax.broadcasted_iota(jnp.int32, (_D, 128), 0)
    col = lax.broadcasted_iota(jnp.int32, (_D, 128), 1)
    es = [(col == row + q * 32).astype(jnp.float32) for q in range(4)]
    c = pl.program_id(0)
    v3 = 3 * _S + c * _VC + lax.broadcasted_iota(jnp.int32, (_D, _VC), 1)
    ok3 = v3 < _V
    refs = (i0, i1, i2, i3)
    for t in range(13):
        acc = None
        for q in range(4):
            x = refs[q][t]
            if q == 3:
                x = jnp.where(ok3, x, 0.0)
            y = lax.dot_general(x, es[q], (((0,), (0,)), ((), ())),
                                preferred_element_type=jnp.float32)
            acc = y if acc is None else acc + y
        out_ref[t] = acc


def _relayout_table(tt3):
    spec = lambda q: pl.BlockSpec(
        (13, _D, _VC), lambda c, g, q=q: (g, 0, q * _NCQ + c))
    out = pl.pallas_call(
        _tr_body,
        grid=(_NCQ, 2),
        in_specs=[spec(0), spec(1), spec(2), spec(3)],
        out_specs=pl.BlockSpec((13, _VC, 128), lambda c, g: (g, c, 0)),
        out_shape=jax.ShapeDtypeStruct((_NT, _S, 128), jnp.float32),
        compiler_params=pltpu.CompilerParams(fuse_transposed_lhs_in_matmul=True),
    )(tt3, tt3, tt3, tt3)
    return out.reshape(_NT * _S, 128)


def _sc_body(table_hbm, idx_hbm, out_hbm, idx_v, buf0, buf1, sem, psem):
    wid = lax.axis_index("s") * 2 + lax.axis_index("c")
    base = wid * _PER_W
    pltpu.sync_copy(idx_hbm.at[wid], idx_v)
    bufs = (buf0, buf1)
    puts = [None, None]
    for j in range(_NCHUNK):
        s = j % 2
        if puts[s] is not None:
            puts[s].wait()
        pltpu.async_copy(table_hbm.at[idx_v.at[j]], bufs[s], sem).wait()
        puts[s] = pltpu.async_copy(
            bufs[s], out_hbm.at[pl.ds(base + j * _CHUNK, _CHUNK)], psem
        )
    for s in range(2):
        if puts[s] is not None:
            puts[s].wait()


def _sc_gather(table4, gidx_grouped):
    mesh = plsc.VectorSubcoreMesh(core_axis_name="c", subcore_axis_name="s")
    k = functools.partial(
        pl.kernel,
        mesh=mesh,
        out_type=jax.ShapeDtypeStruct((_NT * _B, 128), jnp.float32),
        scratch_types=[
            pltpu.VMEM((_NCHUNK, _CHUNK), jnp.int32),
            pltpu.VMEM((_CHUNK, 128), jnp.float32),
            pltpu.VMEM((_CHUNK, 128), jnp.float32),
            pltpu.SemaphoreType.DMA,
            pltpu.SemaphoreType.DMA,
        ],
        compiler_params=pltpu.CompilerParams(use_tc_tiling_on_sc=True),
    )(_sc_body)
    return k(table4, gidx_grouped)


_BLK = 512
_GRID = _B // _BLK

_DNUM_T = (((1,), (1,)), ((), ()))
_DNUM = (((1,), (0,)), ((), ()))


def _mm_t(w, x):
    return lax.dot_general(w, x, _DNUM_T, preferred_element_type=jnp.float32)


def _mm(w, x):
    return lax.dot_general(w, x, _DNUM, preferred_element_type=jnp.float32)


def _fw_body(dense_ref, grp_ref, qsel_ref,
             bw0, bb0, bw1, bb1, bw2, bb2,
             tw0, tb0, tw1, tb1, tw2, tb2,
             out_ref):
    x = dense_ref[...]
    h = jnp.maximum(_mm_t(bw0[...], x) + bb0[...], 0.0)
    h = jnp.maximum(_mm(bw1[...], h) + bb1[...], 0.0)
    xb = jnp.maximum(_mm(bw2[...], h) + bb2[...], 0.0)

    grpt = jnp.transpose(grp_ref[...], (0, 2, 1))
    qsel = qsel_ref[...]
    lyt = grpt[:, 0:32, :]
    for q in range(1, 4):
        m = (qsel == q)[:, None, :]
        lyt = jnp.where(m, grpt[:, q * 32:(q + 1) * 32, :], lyt)
    a = jnp.concatenate([xb[None], lyt], axis=0)

    zs = []
    for i in range(1, _NI):
        prod = a[:i] * a[i][None]
        zs.append(jnp.sum(prod, axis=1))
    r = jnp.concatenate([xb] + zs, axis=0)

    t = jnp.maximum(_mm(tw0[...], r) + tb0[...], 0.0)
    t = jnp.maximum(_mm(tw1[...], t) + tb1[...], 0.0)
    t = _mm(tw2[...], t) + tb2[...]
    out_ref[...] = jax.nn.sigmoid(t)


def _tc_forward(dense_x, grp, qsel, bots, tops):
    full = lambda shape: pl.BlockSpec(shape, lambda i: tuple(0 for _ in shape))
    in_specs = [
        pl.BlockSpec((_BLK, 13), lambda i: (i, 0)),
        pl.BlockSpec((_NT, _BLK, 128), lambda i: (0, i, 0)),
        pl.BlockSpec((_NT, _BLK), lambda i: (0, i)),
    ]
    args = [dense_x, grp, qsel]
    for w, b in bots + tops:
        in_specs += [full(w.shape), full((b.shape[0], 1))]
        args += [w, b.reshape(-1, 1)]
    out = pl.pallas_call(
        _fw_body,
        grid=(_GRID,),
        in_specs=in_specs,
        out_specs=pl.BlockSpec((1, _BLK), lambda i: (0, i)),
        out_shape=jax.ShapeDtypeStruct((1, _B), jnp.float32),
    )(*args)
    return out.reshape(_B, 1)


def kernel(dense_x, lS_o, lS_i, emb_tables,
           bot_w0, bot_b0, bot_w1, bot_b1, bot_w2, bot_b2,
           top_w0, top_b0, top_w1, top_b1, top_w2, top_b2):
    del lS_o
    tt3 = emb_tables.transpose(0, 2, 1)
    table4 = _relayout_table(tt3)
    qsel = lS_i // _S
    gidx = (jnp.arange(_NT, dtype=jnp.int32) * _S)[:, None] + lS_i % _S
    gidx_grouped = gidx.reshape(_NW, _NCHUNK, _CHUNK)
    grp = _sc_gather(table4, gidx_grouped)
    grp3 = grp.reshape(_NT, _B, 128)
    bots = [(bot_w0, bot_b0), (bot_w1, bot_b1), (bot_w2, bot_b2)]
    tops = [(top_w0, top_b0), (top_w1, top_b1), (top_w2, top_b2)]
    return _tc_forward(dense_x, grp3, qsel, bots, tops)


# speedup vs baseline: 1.7445x; 1.0107x over previous
import functools

import jax
import jax.numpy as jnp
import numpy as np
from jax import lax
from jax.experimental import pallas as pl
from jax.experimental.pallas import tpu as pltpu
from jax.experimental.pallas import tpu_sc as plsc

_B = 4096
_NT = 26
_V = 100000
_D = 32
_NI = _NT + 1

_S = 25088
_VC = 896
_NCQ = _S // _VC

_NW = 32
_PER_W = _NT * _B // _NW
_CHUNK = 128
_NCHUNK = _PER_W // _CHUNK


def _tr_body(i0, i1, i2, i3, out_ref):
    row = lax.broadcasted_iota(jnp.int32, (_D, 128), 0)
    col = lax.broadcasted_iota(jnp.int32, (_D, 128), 1)
    es = [(col == row + q * 32).astype(jnp.float32) for q in range(4)]
    c = pl.program_id(0)
    v3 = 3 * _S + c * _VC + lax.broadcasted_iota(jnp.int32, (_D, _VC), 1)
    ok3 = v3 < _V
    refs = (i0, i1, i2, i3)
    for t in range(_NT):
        acc = None
        for q in range(4):
            x = refs[q][t]
            if q == 3:
                x = jnp.where(ok3, x, 0.0)
            y = lax.dot_general(x, es[q], (((0,), (0,)), ((), ())),
                                preferred_element_type=jnp.float32)
            acc = y if acc is None else acc + y
        out_ref[t] = acc


def _relayout_table(tt3):
    spec = lambda q: pl.BlockSpec(
        (_NT, _D, _VC), lambda c, q=q: (0, 0, q * _NCQ + c))
    out = pl.pallas_call(
        _tr_body,
        grid=(_NCQ,),
        in_specs=[spec(0), spec(1), spec(2), spec(3)],
        out_specs=pl.BlockSpec((_NT, _VC, 128), lambda c: (0, c, 0)),
        out_shape=jax.ShapeDtypeStruct((_NT, _S, 128), jnp.float32),
        compiler_params=pltpu.CompilerParams(fuse_transposed_lhs_in_matmul=True),
    )(tt3, tt3, tt3, tt3)
    return out.reshape(_NT * _S, 128)


def _sc_body(table_hbm, idx_hbm, out_hbm, idx_v, buf0, buf1, sem, psem):
    wid = lax.axis_index("s") * 2 + lax.axis_index("c")
    base = wid * _PER_W
    pltpu.sync_copy(idx_hbm.at[wid], idx_v)
    bufs = (buf0, buf1)
    puts = [None, None]
    for j in range(_NCHUNK):
        s = j % 2
        if puts[s] is not None:
            puts[s].wait()
        pltpu.async_copy(table_hbm.at[idx_v.at[j]], bufs[s], sem).wait()
        puts[s] = pltpu.async_copy(
            bufs[s], out_hbm.at[pl.ds(base + j * _CHUNK, _CHUNK)], psem
        )
    for s in range(2):
        if puts[s] is not None:
            puts[s].wait()


def _sc_gather(table4, gidx_grouped):
    mesh = plsc.VectorSubcoreMesh(core_axis_name="c", subcore_axis_name="s")
    k = functools.partial(
        pl.kernel,
        mesh=mesh,
        out_type=jax.ShapeDtypeStruct((_NT * _B, 128), jnp.float32),
        scratch_types=[
            pltpu.VMEM((_NCHUNK, _CHUNK), jnp.int32),
            pltpu.VMEM((_CHUNK, 128), jnp.float32),
            pltpu.VMEM((_CHUNK, 128), jnp.float32),
            pltpu.SemaphoreType.DMA,
            pltpu.SemaphoreType.DMA,
        ],
        compiler_params=pltpu.CompilerParams(use_tc_tiling_on_sc=True),
    )(_sc_body)
    return k(table4, gidx_grouped)


_BLK = 512
_GRID = _B // _BLK

_DNUM_T = (((1,), (1,)), ((), ()))
_DNUM = (((1,), (0,)), ((), ()))


def _mm_t(w, x):
    return lax.dot_general(w, x, _DNUM_T, preferred_element_type=jnp.float32)


def _mm(w, x):
    return lax.dot_general(w, x, _DNUM, preferred_element_type=jnp.float32)


def _fw_body(dense_ref, grp_ref, qsel_ref,
             bw0, bb0, bw1, bb1, bw2, bb2,
             tw0, tb0, tw1, tb1, tw2, tb2,
             out_ref):
    x = dense_ref[...]
    h = jnp.maximum(_mm_t(bw0[...], x) + bb0[...], 0.0)
    h = jnp.maximum(_mm(bw1[...], h) + bb1[...], 0.0)
    xb = jnp.maximum(_mm(bw2[...], h) + bb2[...], 0.0)

    grpt = jnp.transpose(grp_ref[...], (0, 2, 1))
    qsel = qsel_ref[...]
    lyt = grpt[:, 0:32, :]
    for q in range(1, 4):
        m = (qsel == q)[:, None, :]
        lyt = jnp.where(m, grpt[:, q * 32:(q + 1) * 32, :], lyt)
    a = jnp.concatenate([xb[None], lyt], axis=0)

    zs = []
    for i in range(1, _NI):
        prod = a[:i] * a[i][None]
        zs.append(jnp.sum(prod, axis=1))
    r = jnp.concatenate([xb] + zs, axis=0)

    t = jnp.maximum(_mm(tw0[...], r) + tb0[...], 0.0)
    t = jnp.maximum(_mm(tw1[...], t) + tb1[...], 0.0)
    t = _mm(tw2[...], t) + tb2[...]
    out_ref[...] = jax.nn.sigmoid(t)


def _tc_forward(dense_x, grp, qsel, bots, tops):
    full = lambda shape: pl.BlockSpec(shape, lambda i: tuple(0 for _ in shape))
    in_specs = [
        pl.BlockSpec((_BLK, 13), lambda i: (i, 0)),
        pl.BlockSpec((_NT, _BLK, 128), lambda i: (0, i, 0)),
        pl.BlockSpec((_NT, _BLK), lambda i: (0, i)),
    ]
    args = [dense_x, grp, qsel]
    for w, b in bots + tops:
        in_specs += [full(w.shape), full((b.shape[0], 1))]
        args += [w, b.reshape(-1, 1)]
    out = pl.pallas_call(
        _fw_body,
        grid=(_GRID,),
        in_specs=in_specs,
        out_specs=pl.BlockSpec((1, _BLK), lambda i: (0, i)),
        out_shape=jax.ShapeDtypeStruct((1, _B), jnp.float32),
    )(*args)
    return out.reshape(_B, 1)


def kernel(dense_x, lS_o, lS_i, emb_tables,
           bot_w0, bot_b0, bot_w1, bot_b1, bot_w2, bot_b2,
           top_w0, top_b0, top_w1, top_b1, top_w2, top_b2):
    del lS_o
    tt3 = emb_tables.transpose(0, 2, 1)
    table4 = _relayout_table(tt3)
    qsel = lS_i // _S
    gidx = (jnp.arange(_NT, dtype=jnp.int32) * _S)[:, None] + lS_i % _S
    gidx_grouped = gidx.reshape(_NW, _NCHUNK, _CHUNK)
    grp = _sc_gather(table4, gidx_grouped)
    grp3 = grp.reshape(_NT, _B, 128)
    bots = [(bot_w0, bot_b0), (bot_w1, bot_b1), (bot_w2, bot_b2)]
    tops = [(top_w0, top_b0), (top_w1, top_b1), (top_w2, top_b2)]
    return _tc_forward(dense_x, grp3, qsel, bots, tops)
